# Initial kernel scaffold; baseline (speedup 1.0000x reference)
import jax
import jax.numpy as jnp
from jax.experimental import pallas as pl


def kernel(sample_points, closest_points, planes, axes, bound, grid_size):
    raise NotImplementedError("write your pallas kernel here")


# speedup vs baseline: 171.1073x; 171.1073x over previous
import functools

import jax
import jax.numpy as jnp
from jax import lax
from jax.experimental import pallas as pl
from jax.experimental.pallas import tpu as pltpu
from jax.experimental.pallas import tpu_sc as plsc

_L = 16
_NC = 2
_NS = 16
_NW = _NC * _NS


def _newton_sqrt(s):
    b = lax.bitcast_convert_type(s, jnp.int32)
    g = lax.bitcast_convert_type((b >> 1) + jnp.int32(0x1FBD1DF6), jnp.float32)
    g = 0.5 * (g + s / g)
    g = 0.5 * (g + s / g)
    g = 0.5 * (g + s / g)
    return g


def _make_sc_call(npts, nt, tsize, gsize, pts_per_w):
    vregs_per_w = pts_per_w // _L
    fg = float(gsize)
    fg1 = float(gsize - 1)

    def body(tp_hbm, xs_hbm, ys_hbm, zs_hbm, table_hbm, out_hbm,
             tp_v, x_v, y_v, z_v, table_v, res_v):
        cid = lax.axis_index("c")
        sid = lax.axis_index("s")
        wid = sid * _NC + cid
        base = wid * pts_per_w

        pltpu.sync_copy(table_hbm, table_v)
        pltpu.sync_copy(tp_hbm, tp_v)
        pltpu.sync_copy(xs_hbm.at[pl.ds(base, pts_per_w)], x_v)
        pltpu.sync_copy(ys_hbm.at[pl.ds(base, pts_per_w)], y_v)
        pltpu.sync_copy(zs_hbm.at[pl.ds(base, pts_per_w)], z_v)

        nv = lax.min(vregs_per_w, lax.max(0, (npts - base) // _L))

        acc = jnp.zeros((_L,), jnp.float32)
        for t in range(nt):
            row = tp_v[t]
            a00 = row[0]
            a01 = row[1]
            a02 = row[2]
            a10 = row[3]
            a11 = row[4]
            a12 = row[5]
            a20 = row[6]
            a21 = row[7]
            a22 = row[8]
            b0 = row[9]
            b1 = row[10]
            b2 = row[11]
            gb = row[12]

            def step(j, acc, _t=t, _a=(a00, a01, a02, a10, a11, a12,
                                        a20, a21, a22, b0, b1, b2, gb)):
                (a00, a01, a02, a10, a11, a12,
                 a20, a21, a22, b0, b1, b2, gb) = _a
                x = x_v[pl.ds(j * _L, _L)]
                y = y_v[pl.ds(j * _L, _L)]
                z = z_v[pl.ds(j * _L, _L)]
                px = a00 * x + a01 * y + a02 * z + b0
                py = a10 * x + a11 * y + a12 * z + b1
                pz = a20 * x + a21 * y + a22 * z + b2
                fx = jnp.minimum(jnp.maximum(px * fg + gb, 0.0), fg1)
                fy = jnp.minimum(jnp.maximum(py * fg + gb, 0.0), fg1)
                fz = jnp.minimum(jnp.maximum(pz * fg + gb, 0.0), fg1)
                ix = fx.astype(jnp.int32)
                iy = fy.astype(jnp.int32)
                iz = fz.astype(jnp.int32)
                g = (ix << 10) + (iy << 5) + iz
                cx = plsc.load_gather(table_v, [g])
                cy = plsc.load_gather(table_v, [g + tsize])
                cz = plsc.load_gather(table_v, [g + 2 * tsize])
                dx = px - cx
                dy = py - cy
                dz = pz - cz
                s = dx * dx + dy * dy + dz * dz
                return acc + _newton_sqrt(s)

            acc = lax.fori_loop(0, nv, step, acc)

        res_v[...] = acc
        pltpu.sync_copy(res_v, out_hbm.at[wid])

    mesh = plsc.VectorSubcoreMesh(core_axis_name="c", subcore_axis_name="s")
    return pl.kernel(
        body,
        out_type=jax.ShapeDtypeStruct((_NW, _L), jnp.float32),
        mesh=mesh,
        compiler_params=pltpu.CompilerParams(needs_layout_passes=False),
        scratch_types=[
            pltpu.VMEM((nt, _L), jnp.float32),
            pltpu.VMEM((pts_per_w,), jnp.float32),
            pltpu.VMEM((pts_per_w,), jnp.float32),
            pltpu.VMEM((pts_per_w,), jnp.float32),
            pltpu.VMEM((3 * tsize,), jnp.float32),
            pltpu.VMEM((_L,), jnp.float32),
        ],
    )


def kernel(sample_points, closest_points, planes, axes, bound, grid_size):
    pts = sample_points.reshape(-1, 3)
    npts = pts.shape[0]
    gsize = closest_points.shape[0]
    tsize = gsize * gsize * gsize

    n = planes[:, :3]
    d = planes[:, 3]
    nn = jnp.sum(n * n, axis=1)
    a_ref = (jnp.eye(3, dtype=jnp.float32)[None]
             - 2.0 * n[:, :, None] * n[:, None, :] / nn[:, None, None])
    b_ref = -2.0 * d[:, None] * n / nn[:, None]
    q = axes[:, 1:]
    a_rot = -(q * q)[:, :, None] * jnp.eye(3, dtype=jnp.float32)[None]
    b_rot = jnp.zeros((axes.shape[0], 3), jnp.float32)
    amat = jnp.concatenate([a_ref, a_rot], axis=0).reshape(-1, 9)
    bvec = jnp.concatenate([b_ref, b_rot], axis=0)
    nt = amat.shape[0]
    gb = jnp.full((nt, 1), grid_size * bound, jnp.float32)
    tparams = jnp.concatenate(
        [amat, bvec, gb, jnp.zeros((nt, _L - 13), jnp.float32)], axis=1)

    vregs = -(-npts // _L)
    pts_per_w = -(-vregs // _NW) * _L
    npad = pts_per_w * _NW
    xs = jnp.pad(pts[:, 0], (0, npad - npts))
    ys = jnp.pad(pts[:, 1], (0, npad - npts))
    zs = jnp.pad(pts[:, 2], (0, npad - npts))
    table = closest_points.reshape(tsize, 3).T.reshape(-1)

    call = _make_sc_call(npts, nt, tsize, gsize, pts_per_w)
    partials = call(tparams, xs, ys, zs, table)
    return jnp.sum(partials).reshape(1)


# speedup vs baseline: 185.2446x; 1.0826x over previous
import functools

import jax
import jax.numpy as jnp
from jax import lax
from jax.experimental import pallas as pl
from jax.experimental.pallas import tpu as pltpu
from jax.experimental.pallas import tpu_sc as plsc

_L = 16
_NC = 2
_NS = 16
_NW = _NC * _NS


def _norm16(s):
    s = jnp.maximum(s, jnp.float32(1e-25))
    b = lax.bitcast_convert_type(s, jnp.int32)
    y = lax.bitcast_convert_type(jnp.int32(0x5F3759DF) - (b >> 1), jnp.float32)
    sh = 0.5 * s
    y = y * (1.5 - sh * y * y)
    y = y * (1.5 - sh * y * y)
    return s * y


def _make_sc_call(npts, ngen, nt, tsize, gsize, pts_per_w):
    vregs_per_w = pts_per_w // _L
    fg = float(gsize)
    fg1 = float(gsize - 1)
    shift = gsize.bit_length() - 1

    def body(tp_hbm, xs_hbm, ys_hbm, zs_hbm, table_hbm, out_hbm,
             tp_v, x_v, y_v, z_v, table_v, res_v):
        cid = lax.axis_index("c")
        sid = lax.axis_index("s")
        wid = sid * _NC + cid
        base = wid * pts_per_w

        pltpu.sync_copy(table_hbm, table_v)
        pltpu.sync_copy(tp_hbm, tp_v)
        pltpu.sync_copy(xs_hbm.at[pl.ds(base, pts_per_w)], x_v)
        pltpu.sync_copy(ys_hbm.at[pl.ds(base, pts_per_w)], y_v)
        pltpu.sync_copy(zs_hbm.at[pl.ds(base, pts_per_w)], z_v)

        nv = lax.min(vregs_per_w, lax.max(0, (npts - base) // _L))

        acc = jnp.zeros((_L,), jnp.float32)
        for t in range(nt):
            row = tp_v[t]
            a00 = row[0]
            a11 = row[4]
            a22 = row[8]
            gb = row[12]
            if t < ngen:
                a01 = row[1]
                a02 = row[2]
                a10 = row[3]
                a12 = row[5]
                a20 = row[6]
                a21 = row[7]
                b0 = row[9]
                b1 = row[10]
                b2 = row[11]

            def step(j, acc, _t=t):
                x = x_v[pl.ds(j * _L, _L)]
                y = y_v[pl.ds(j * _L, _L)]
                z = z_v[pl.ds(j * _L, _L)]
                if _t < ngen:
                    px = a00 * x + a01 * y + a02 * z + b0
                    py = a10 * x + a11 * y + a12 * z + b1
                    pz = a20 * x + a21 * y + a22 * z + b2
                else:
                    px = a00 * x
                    py = a11 * y
                    pz = a22 * z
                fx = jnp.minimum(jnp.maximum(px * fg + gb, 0.0), fg1)
                fy = jnp.minimum(jnp.maximum(py * fg + gb, 0.0), fg1)
                fz = jnp.minimum(jnp.maximum(pz * fg + gb, 0.0), fg1)
                ix = fx.astype(jnp.int32)
                iy = fy.astype(jnp.int32)
                iz = fz.astype(jnp.int32)
                g = (ix << (2 * shift)) + (iy << shift) + iz
                cx = plsc.load_gather(table_v, [g])
                cy = plsc.load_gather(table_v, [g + tsize])
                cz = plsc.load_gather(table_v, [g + 2 * tsize])
                dx = px - cx
                dy = py - cy
                dz = pz - cz
                return acc + _norm16(dx * dx + dy * dy + dz * dz)

            acc = plsc.parallel_loop(0, nv, unroll=4, carry=acc)(step)

        res_v[...] = acc
        pltpu.sync_copy(res_v, out_hbm.at[wid])

    mesh = plsc.VectorSubcoreMesh(core_axis_name="c", subcore_axis_name="s")
    return pl.kernel(
        body,
        out_type=jax.ShapeDtypeStruct((_NW, _L), jnp.float32),
        mesh=mesh,
        compiler_params=pltpu.CompilerParams(needs_layout_passes=False),
        scratch_types=[
            pltpu.VMEM((nt, _L), jnp.float32),
            pltpu.VMEM((pts_per_w,), jnp.float32),
            pltpu.VMEM((pts_per_w,), jnp.float32),
            pltpu.VMEM((pts_per_w,), jnp.float32),
            pltpu.VMEM((3 * tsize,), jnp.float32),
            pltpu.VMEM((_L,), jnp.float32),
        ],
    )


def kernel(sample_points, closest_points, planes, axes, bound, grid_size):
    pts = sample_points.reshape(-1, 3)
    npts = pts.shape[0]
    gsize = closest_points.shape[0]
    tsize = gsize * gsize * gsize

    n = planes[:, :3]
    d = planes[:, 3]
    nn = jnp.sum(n * n, axis=1)
    a_ref = (jnp.eye(3, dtype=jnp.float32)[None]
             - 2.0 * n[:, :, None] * n[:, None, :] / nn[:, None, None])
    b_ref = -2.0 * d[:, None] * n / nn[:, None]
    q = axes[:, 1:]
    a_rot = -(q * q)[:, :, None] * jnp.eye(3, dtype=jnp.float32)[None]
    b_rot = jnp.zeros((axes.shape[0], 3), jnp.float32)
    amat = jnp.concatenate([a_ref, a_rot], axis=0).reshape(-1, 9)
    bvec = jnp.concatenate([b_ref, b_rot], axis=0)
    ngen = planes.shape[0]
    nt = amat.shape[0]
    gb = jnp.full((nt, 1), grid_size * bound, jnp.float32)
    tparams = jnp.concatenate(
        [amat, bvec, gb, jnp.zeros((nt, _L - 13), jnp.float32)], axis=1)

    vregs = -(-npts // _L)
    pts_per_w = -(-vregs // _NW) * _L
    npad = pts_per_w * _NW
    xs = jnp.pad(pts[:, 0], (0, npad - npts))
    ys = jnp.pad(pts[:, 1], (0, npad - npts))
    zs = jnp.pad(pts[:, 2], (0, npad - npts))
    table = closest_points.reshape(tsize, 3).T.reshape(-1)

    call = _make_sc_call(npts, ngen, nt, tsize, gsize, pts_per_w)
    partials = call(tparams, xs, ys, zs, table)
    return jnp.sum(partials).reshape(1)
